# baseline (speedup 1.0000x reference)
import functools
import math

import jax
import jax.numpy as jnp
from jax import lax
from jax.experimental import pallas as pl
from jax.experimental.pallas import tpu as pltpu
from jax.experimental.pallas import tpu_sc as plsc

N_NODES = 10000
N_EDGES = 320000
IN_C = 128
HID = 128
HHID = HID // 2
OUT_C = 64
NUM_LAYERS = 16
ALPHA = 0.1
LMBDA = 0.5
BETAS = [math.log(LMBDA / (i + 1) + 1.0) for i in range(NUM_LAYERS)]

NC = 2
NS = 16
CHUNK = 128
EPT = -(-N_EDGES // NS)
NCHUNK = -(-EPT // CHUNK)
NCHUNK = NCHUNK + (NCHUNK % 2)
EPT_PAD = NCHUNK * CHUNK
E_PAD = NS * EPT_PAD
TRASH = N_NODES
ACC_ROWS = N_NODES + 8
RPT = 624
REM = N_NODES - NS * RPT
REMZ = ACC_ROWS - NS * RPT

BN = 2000
NB = N_NODES // BN

_MESH = plsc.VectorSubcoreMesh(core_axis_name="c", subcore_axis_name="s")
_DOT11 = (((1,), (1,)), ((), ()))


def _sc_degree(rows3, ones_hbm, zeros_hbm):
    @functools.partial(
        pl.kernel,
        out_type=jax.ShapeDtypeStruct((NC, N_NODES, 16), jnp.float32),
        mesh=_MESH,
        scratch_types=[
            pltpu.VMEM((NCHUNK, CHUNK), jnp.int32),
            pltpu.VMEM((CHUNK, 16), jnp.float32),
            pltpu.VMEM_SHARED((ACC_ROWS, 16), jnp.float32),
        ],
        compiler_params=pltpu.CompilerParams(use_tc_tiling_on_sc=False),
    )
    def k(rows_hbm, ones_h, z_hbm, out_hbm, rows_v, ones_v, acc):
        cid = lax.axis_index("c")
        sid = lax.axis_index("s")
        pltpu.sync_copy(rows_hbm.at[sid], rows_v)
        pltpu.sync_copy(ones_h, ones_v)
        base = sid * RPT
        pltpu.sync_copy(z_hbm.at[pl.ds(0, RPT)], acc.at[pl.ds(base, RPT)])

        @pl.when(sid == NS - 1)
        def _():
            pltpu.sync_copy(z_hbm.at[pl.ds(0, REMZ)],
                            acc.at[pl.ds(NS * RPT, REMZ)])

        plsc.subcore_barrier()

        @pl.loop(0, NCHUNK)
        def _(j):
            pltpu.sync_copy(ones_v, acc.at[rows_v.at[j]], add=True)

        plsc.subcore_barrier()
        pltpu.sync_copy(acc.at[pl.ds(base, RPT)],
                        out_hbm.at[cid].at[pl.ds(base, RPT)])

        @pl.when(sid == NS - 1)
        def _():
            pltpu.sync_copy(acc.at[pl.ds(NS * RPT, REM)],
                            out_hbm.at[cid].at[pl.ds(NS * RPT, REM)])

    return k(rows3, ones_hbm, zeros_hbm)


def _sc_spmm(u3, rows3, cols3, zeros_hbm):
    @functools.partial(
        pl.kernel,
        out_type=jax.ShapeDtypeStruct((NC, N_NODES, HHID), jnp.float32),
        mesh=_MESH,
        scratch_types=[
            pltpu.VMEM((NCHUNK, CHUNK), jnp.int32),
            pltpu.VMEM((NCHUNK, CHUNK), jnp.int32),
            pltpu.VMEM((CHUNK, HHID), jnp.float32),
            pltpu.VMEM((CHUNK, HHID), jnp.float32),
            pltpu.VMEM_SHARED((ACC_ROWS, HHID), jnp.float32),
            pltpu.SemaphoreType.DMA,
            pltpu.SemaphoreType.DMA,
        ],
        compiler_params=pltpu.CompilerParams(use_tc_tiling_on_sc=False),
    )
    def k(u_hbm, rows_hbm, cols_hbm, z_hbm, out_hbm,
          rows_v, cols_v, buf_a, buf_b, acc, sem_a, sem_b):
        cid = lax.axis_index("c")
        sid = lax.axis_index("s")
        pltpu.sync_copy(rows_hbm.at[sid], rows_v)
        pltpu.sync_copy(cols_hbm.at[sid], cols_v)
        base = sid * RPT
        pltpu.sync_copy(z_hbm.at[pl.ds(0, RPT)], acc.at[pl.ds(base, RPT)])

        @pl.when(sid == NS - 1)
        def __():
            pltpu.sync_copy(z_hbm.at[pl.ds(0, REMZ)],
                            acc.at[pl.ds(NS * RPT, REMZ)])

        plsc.subcore_barrier()

        @pl.loop(0, NCHUNK, step=2)
        def _(j):
            ca = pltpu.async_copy(u_hbm.at[cid].at[cols_v.at[j]], buf_a, sem_a)
            cb = pltpu.async_copy(u_hbm.at[cid].at[cols_v.at[j + 1]], buf_b,
                                  sem_b)
            ca.wait()
            pltpu.sync_copy(buf_a, acc.at[rows_v.at[j]], add=True)
            cb.wait()
            pltpu.sync_copy(buf_b, acc.at[rows_v.at[j + 1]], add=True)

        plsc.subcore_barrier()
        pltpu.sync_copy(acc.at[pl.ds(base, RPT)],
                        out_hbm.at[cid].at[pl.ds(base, RPT)])

        @pl.when(sid == NS - 1)
        def _():
            pltpu.sync_copy(acc.at[pl.ds(NS * RPT, REM)],
                            out_hbm.at[cid].at[pl.ds(NS * RPT, REM)])

    return k(u3, rows3, cols3, zeros_hbm)


def _full(shape):
    return pl.BlockSpec(shape, lambda b: tuple(0 for _ in shape))


def _rows(width=HID):
    return pl.BlockSpec((BN, width), lambda b: (b, 0))


def _phalf_spec():
    return pl.BlockSpec((NC, BN, HHID), lambda b: (0, b, 0))


def _join(p_ref):
    return jnp.concatenate([p_ref[0], p_ref[1]], axis=1)


def _tc_prep(x, W0, b0):
    def body(x_ref, w_ref, b_ref, g_ref):
        xw = lax.dot_general(x_ref[...], w_ref[...], _DOT11,
                             preferred_element_type=jnp.float32)
        g_ref[...] = jax.nn.relu(xw + b_ref[...])

    return pl.pallas_call(
        body,
        grid=(NB,),
        in_specs=[_rows(IN_C), _full((HID, IN_C)), _full((1, HID))],
        out_specs=_rows(),
        out_shape=jax.ShapeDtypeStruct((N_NODES, HID), jnp.float32),
    )(x, W0, b0.reshape(1, HID))


def _tc_dis(deg_p):
    def body(d_ref, o_ref):
        d = d_ref[0]
        d128 = jnp.concatenate([d] * (HID // 16), axis=1)
        o_ref[...] = jnp.where(d128 > 0.0,
                               lax.rsqrt(jnp.maximum(d128, 1.0)), 0.0)

    return pl.pallas_call(
        body,
        grid=(NB,),
        in_specs=[pl.BlockSpec((1, BN, 16), lambda b: (0, b, 0))],
        out_specs=_rows(),
        out_shape=jax.ShapeDtypeStruct((N_NODES, HID), jnp.float32),
    )(deg_p)


def _tc_cterms(g, Wb, sgb, bb):
    def body(g_ref, w_ref, s_ref, b_ref, c_ref):
        gw = lax.dot_general(g_ref[...].astype(jnp.bfloat16),
                             w_ref[0].astype(jnp.bfloat16), _DOT11,
                             preferred_element_type=jnp.float32)
        c_ref[0] = gw + s_ref[0] * g_ref[...] + b_ref[0]

    return pl.pallas_call(
        body,
        grid=(NUM_LAYERS, NB),
        in_specs=[pl.BlockSpec((BN, HID), lambda i, j: (j, 0)),
                  pl.BlockSpec((1, HID, HID), lambda i, j: (i, 0, 0)),
                  pl.BlockSpec((1, 1, HID), lambda i, j: (i, 0, 0)),
                  pl.BlockSpec((1, 1, HID), lambda i, j: (i, 0, 0))],
        out_specs=pl.BlockSpec((1, BN, HID), lambda i, j: (i, j, 0)),
        out_shape=jax.ShapeDtypeStruct((NUM_LAYERS, N_NODES, HID), jnp.float32),
    )(g, Wb, sgb, bb)


def _uhalves_spec():
    return pl.BlockSpec((NC, BN, HHID), lambda b: (0, b, 0))


def _write_u_halves(u_ref, u):
    u_ref[0] = u[:, :HHID]
    u_ref[1] = u[:, HHID:]


def _tc_u0(g, dis, W0l, beta):
    def body(g_ref, d_ref, w_ref, u_ref):
        gw = lax.dot_general(g_ref[...], w_ref[...], _DOT11,
                             preferred_element_type=jnp.float32)
        _write_u_halves(u_ref,
                        d_ref[...] * (beta * gw + (1.0 - beta) * g_ref[...]))

    return pl.pallas_call(
        body,
        grid=(NB,),
        in_specs=[_rows(), _rows(), _full((HID, HID))],
        out_specs=_uhalves_spec(),
        out_shape=jax.ShapeDtypeStruct((NC, N_NODES, HHID), jnp.float32),
    )(g, dis, W0l)


def _tc_layer(P, dis, Ci, Wn, beta):
    def body(p_ref, d_ref, c_ref, w_ref, u_ref):
        s = _join(p_ref) * d_ref[...]
        h = jax.nn.relu((1.0 - ALPHA) * s + c_ref[...])
        hw = lax.dot_general(h, w_ref[...], _DOT11,
                             preferred_element_type=jnp.float32)
        _write_u_halves(u_ref, d_ref[...] * (beta * hw + (1.0 - beta) * h))

    return pl.pallas_call(
        body,
        grid=(NB,),
        in_specs=[_phalf_spec(), _rows(), _rows(), _full((HID, HID))],
        out_specs=_uhalves_spec(),
        out_shape=jax.ShapeDtypeStruct((NC, N_NODES, HHID), jnp.float32),
    )(P, dis, Ci, Wn)


def _tc_final(P, dis, Ci, W1, b1):
    def body(p_ref, d_ref, c_ref, w_ref, b_ref, o_ref):
        s = _join(p_ref) * d_ref[...]
        h = jax.nn.relu((1.0 - ALPHA) * s + c_ref[...])
        hw = lax.dot_general(h, w_ref[...], _DOT11,
                             preferred_element_type=jnp.float32)
        o_ref[...] = hw + b_ref[...]

    return pl.pallas_call(
        body,
        grid=(NB,),
        in_specs=[_phalf_spec(), _rows(), _rows(),
                  _full((OUT_C, HID)), _full((1, OUT_C))],
        out_specs=_rows(OUT_C),
        out_shape=jax.ShapeDtypeStruct((N_NODES, OUT_C), jnp.float32),
    )(P, dis, Ci, W1, b1.reshape(1, OUT_C))


def kernel(x, edge_index, W0, b0, W1, b1, Wl, bl):
    perm = jnp.argsort(edge_index[0])
    row = edge_index[0][perm]
    col = edge_index[1][perm]
    pad_n = E_PAD - N_EDGES
    rows3 = jnp.concatenate(
        [row, jnp.full((pad_n,), TRASH, jnp.int32)]).reshape(NS, NCHUNK, CHUNK)
    cols3 = jnp.concatenate(
        [col, jnp.zeros((pad_n,), jnp.int32)]).reshape(NS, NCHUNK, CHUNK)
    zeros_h = jnp.zeros((RPT, HHID), jnp.float32)
    zeros16 = jnp.zeros((RPT, 16), jnp.float32)
    ones16 = jnp.ones((CHUNK, 16), jnp.float32)

    beta = jnp.asarray(BETAS, jnp.float32)
    Wb = ALPHA * beta[:, None, None] * Wl
    sgb = jnp.broadcast_to((ALPHA * (1.0 - beta))[:, None, None],
                           (NUM_LAYERS, 1, HID))
    bb = (beta[:, None] * bl)[:, None, :]

    deg_p = _sc_degree(rows3, ones16, zeros16)
    dis = _tc_dis(deg_p)
    g = _tc_prep(x, W0, b0)

    u = _tc_u0(g, dis, Wl[0], BETAS[0])
    P = _sc_spmm(u, rows3, cols3, zeros_h)
    Cstack = _tc_cterms(g, Wb, sgb, bb)
    Cs = [Cstack[i] for i in range(NUM_LAYERS)]
    for i in range(NUM_LAYERS):
        if i > 0:
            P = _sc_spmm(u, rows3, cols3, zeros_h)
        if i + 1 < NUM_LAYERS:
            u = _tc_layer(P, dis, Cs[i], Wl[i + 1], BETAS[i + 1])
        else:
            return _tc_final(P, dis, Cs[i], W1, b1)


# speedup vs baseline: 1.1715x; 1.1715x over previous
import functools
import math

import jax
import jax.numpy as jnp
from jax import lax
from jax.experimental import pallas as pl
from jax.experimental.pallas import tpu as pltpu
from jax.experimental.pallas import tpu_sc as plsc

N_NODES = 10000
N_EDGES = 320000
IN_C = 128
HID = 128
HHID = HID // 2
OUT_C = 64
NUM_LAYERS = 16
ALPHA = 0.1
LMBDA = 0.5
BETAS = [math.log(LMBDA / (i + 1) + 1.0) for i in range(NUM_LAYERS)]

NC = 2
NS = 16
CHUNK = 128
EPT = -(-N_EDGES // NS)
NCHUNK = -(-EPT // CHUNK)
NCHUNK = NCHUNK + (NCHUNK % 2)
EPT_PAD = NCHUNK * CHUNK
E_PAD = NS * EPT_PAD
TRASH = N_NODES
ACC_ROWS = N_NODES + 8
RPT = 624
REM = N_NODES - NS * RPT
REMZ = ACC_ROWS - NS * RPT

BN = 2000
NB = N_NODES // BN

_MESH = plsc.VectorSubcoreMesh(core_axis_name="c", subcore_axis_name="s")
_DOT11 = (((1,), (1,)), ((), ()))


def _sc_degree(rows3, ones_hbm, zeros_hbm):
    @functools.partial(
        pl.kernel,
        out_type=jax.ShapeDtypeStruct((NC, N_NODES, 16), jnp.float32),
        mesh=_MESH,
        scratch_types=[
            pltpu.VMEM((NCHUNK, CHUNK), jnp.int32),
            pltpu.VMEM((CHUNK, 16), jnp.float32),
            pltpu.VMEM_SHARED((ACC_ROWS, 16), jnp.float32),
        ],
        compiler_params=pltpu.CompilerParams(use_tc_tiling_on_sc=False),
    )
    def k(rows_hbm, ones_h, z_hbm, out_hbm, rows_v, ones_v, acc):
        cid = lax.axis_index("c")
        sid = lax.axis_index("s")
        pltpu.sync_copy(rows_hbm.at[sid], rows_v)
        pltpu.sync_copy(ones_h, ones_v)
        base = sid * RPT
        pltpu.sync_copy(z_hbm.at[pl.ds(0, RPT)], acc.at[pl.ds(base, RPT)])

        @pl.when(sid == NS - 1)
        def _():
            pltpu.sync_copy(z_hbm.at[pl.ds(0, REMZ)],
                            acc.at[pl.ds(NS * RPT, REMZ)])

        plsc.subcore_barrier()

        @pl.loop(0, NCHUNK)
        def _(j):
            pltpu.sync_copy(ones_v, acc.at[rows_v.at[j]], add=True)

        plsc.subcore_barrier()
        pltpu.sync_copy(acc.at[pl.ds(base, RPT)],
                        out_hbm.at[cid].at[pl.ds(base, RPT)])

        @pl.when(sid == NS - 1)
        def _():
            pltpu.sync_copy(acc.at[pl.ds(NS * RPT, REM)],
                            out_hbm.at[cid].at[pl.ds(NS * RPT, REM)])

    return k(rows3, ones_hbm, zeros_hbm)


def _sc_spmm(u3, rows3, cols3, zeros_hbm):
    @functools.partial(
        pl.kernel,
        out_type=jax.ShapeDtypeStruct((NC, N_NODES, HHID), jnp.float32),
        mesh=_MESH,
        scratch_types=[
            pltpu.VMEM((NCHUNK, CHUNK), jnp.int32),
            pltpu.VMEM((NCHUNK, CHUNK), jnp.int32),
            pltpu.VMEM((CHUNK, HHID), jnp.float32),
            pltpu.VMEM((CHUNK, HHID), jnp.float32),
            pltpu.VMEM_SHARED((ACC_ROWS, HHID), jnp.float32),
            pltpu.SemaphoreType.DMA,
            pltpu.SemaphoreType.DMA,
        ],
        compiler_params=pltpu.CompilerParams(use_tc_tiling_on_sc=False),
    )
    def k(u_hbm, rows_hbm, cols_hbm, z_hbm, out_hbm,
          rows_v, cols_v, buf_a, buf_b, acc, sem_a, sem_b):
        cid = lax.axis_index("c")
        sid = lax.axis_index("s")
        pltpu.sync_copy(rows_hbm.at[sid], rows_v)
        pltpu.sync_copy(cols_hbm.at[sid], cols_v)
        base = sid * RPT
        pltpu.sync_copy(z_hbm.at[pl.ds(0, RPT)], acc.at[pl.ds(base, RPT)])

        @pl.when(sid == NS - 1)
        def __():
            pltpu.sync_copy(z_hbm.at[pl.ds(0, REMZ)],
                            acc.at[pl.ds(NS * RPT, REMZ)])

        plsc.subcore_barrier()

        @pl.loop(0, NCHUNK, step=2)
        def _(j):
            ca = pltpu.async_copy(u_hbm.at[cid].at[cols_v.at[j]], buf_a, sem_a)
            cb = pltpu.async_copy(u_hbm.at[cid].at[cols_v.at[j + 1]], buf_b,
                                  sem_b)
            ca.wait()
            pltpu.sync_copy(buf_a, acc.at[rows_v.at[j]], add=True)
            cb.wait()
            pltpu.sync_copy(buf_b, acc.at[rows_v.at[j + 1]], add=True)

        plsc.subcore_barrier()
        pltpu.sync_copy(acc.at[pl.ds(base, RPT)],
                        out_hbm.at[cid].at[pl.ds(base, RPT)])

        @pl.when(sid == NS - 1)
        def _():
            pltpu.sync_copy(acc.at[pl.ds(NS * RPT, REM)],
                            out_hbm.at[cid].at[pl.ds(NS * RPT, REM)])

    return k(u3, rows3, cols3, zeros_hbm)


def _full(shape):
    return pl.BlockSpec(shape, lambda b: tuple(0 for _ in shape))


def _rows(width=HID):
    return pl.BlockSpec((BN, width), lambda b: (b, 0))


def _phalf_spec():
    return pl.BlockSpec((NC, BN, HHID), lambda b: (0, b, 0))


def _join(p_ref):
    return jnp.concatenate([p_ref[0], p_ref[1]], axis=1)


def _tc_prep(x, W0, b0):
    def body(x_ref, w_ref, b_ref, g_ref):
        xw = lax.dot_general(x_ref[...], w_ref[...], _DOT11,
                             preferred_element_type=jnp.float32)
        g_ref[...] = jax.nn.relu(xw + b_ref[...])

    return pl.pallas_call(
        body,
        grid=(NB,),
        in_specs=[_rows(IN_C), _full((HID, IN_C)), _full((1, HID))],
        out_specs=_rows(),
        out_shape=jax.ShapeDtypeStruct((N_NODES, HID), jnp.float32),
    )(x, W0, b0.reshape(1, HID))


def _tc_dis(deg_p):
    def body(d_ref, o_ref):
        d = d_ref[0]
        d128 = jnp.concatenate([d] * (HID // 16), axis=1)
        o_ref[...] = jnp.where(d128 > 0.0,
                               lax.rsqrt(jnp.maximum(d128, 1.0)), 0.0)

    return pl.pallas_call(
        body,
        grid=(NB,),
        in_specs=[pl.BlockSpec((1, BN, 16), lambda b: (0, b, 0))],
        out_specs=_rows(),
        out_shape=jax.ShapeDtypeStruct((N_NODES, HID), jnp.float32),
    )(deg_p)


def _tc_cterms(g, Wb, sgb, bb):
    def body(g_ref, w_ref, s_ref, b_ref, c_ref):
        gw = lax.dot_general(g_ref[...].astype(jnp.bfloat16),
                             w_ref[0].astype(jnp.bfloat16), _DOT11,
                             preferred_element_type=jnp.float32)
        c_ref[0] = gw + s_ref[0] * g_ref[...] + b_ref[0]

    return pl.pallas_call(
        body,
        grid=(NUM_LAYERS, NB),
        in_specs=[pl.BlockSpec((BN, HID), lambda i, j: (j, 0)),
                  pl.BlockSpec((1, HID, HID), lambda i, j: (i, 0, 0)),
                  pl.BlockSpec((1, 1, HID), lambda i, j: (i, 0, 0)),
                  pl.BlockSpec((1, 1, HID), lambda i, j: (i, 0, 0))],
        out_specs=pl.BlockSpec((1, BN, HID), lambda i, j: (i, j, 0)),
        out_shape=jax.ShapeDtypeStruct((NUM_LAYERS, N_NODES, HID), jnp.float32),
    )(g, Wb, sgb, bb)


def _uhalves_spec():
    return pl.BlockSpec((NC, BN, HHID), lambda b: (0, b, 0))


def _write_u_halves(u_ref, u):
    u_ref[0] = u[:, :HHID]
    u_ref[1] = u[:, HHID:]


def _tc_u0(g, dis, W0l, beta):
    def body(g_ref, d_ref, w_ref, u_ref):
        gw = lax.dot_general(g_ref[...], w_ref[...], _DOT11,
                             preferred_element_type=jnp.float32)
        _write_u_halves(u_ref,
                        d_ref[...] * (beta * gw + (1.0 - beta) * g_ref[...]))

    return pl.pallas_call(
        body,
        grid=(NB,),
        in_specs=[_rows(), _rows(), _full((HID, HID))],
        out_specs=_uhalves_spec(),
        out_shape=jax.ShapeDtypeStruct((NC, N_NODES, HHID), jnp.float32),
    )(g, dis, W0l)


def _tc_layer(P, dis, Ci, Wn, beta):
    def body(p_ref, d_ref, c_ref, w_ref, u_ref):
        s = _join(p_ref) * d_ref[...]
        h = jax.nn.relu((1.0 - ALPHA) * s + c_ref[...])
        hw = lax.dot_general(h, w_ref[...], _DOT11,
                             preferred_element_type=jnp.float32)
        _write_u_halves(u_ref, d_ref[...] * (beta * hw + (1.0 - beta) * h))

    return pl.pallas_call(
        body,
        grid=(NB,),
        in_specs=[_phalf_spec(), _rows(), _rows(), _full((HID, HID))],
        out_specs=_uhalves_spec(),
        out_shape=jax.ShapeDtypeStruct((NC, N_NODES, HHID), jnp.float32),
    )(P, dis, Ci, Wn)


def _tc_final(P, dis, Ci, W1, b1):
    def body(p_ref, d_ref, c_ref, w_ref, b_ref, o_ref):
        s = _join(p_ref) * d_ref[...]
        h = jax.nn.relu((1.0 - ALPHA) * s + c_ref[...])
        hw = lax.dot_general(h, w_ref[...], _DOT11,
                             preferred_element_type=jnp.float32)
        o_ref[...] = hw + b_ref[...]

    return pl.pallas_call(
        body,
        grid=(NB,),
        in_specs=[_phalf_spec(), _rows(), _rows(),
                  _full((OUT_C, HID)), _full((1, OUT_C))],
        out_specs=_rows(OUT_C),
        out_shape=jax.ShapeDtypeStruct((N_NODES, OUT_C), jnp.float32),
    )(P, dis, Ci, W1, b1.reshape(1, OUT_C))


def kernel(x, edge_index, W0, b0, W1, b1, Wl, bl):
    row = edge_index[0]
    col = edge_index[1]
    pad_n = E_PAD - N_EDGES
    rows3 = jnp.concatenate(
        [row, jnp.full((pad_n,), TRASH, jnp.int32)]).reshape(NS, NCHUNK, CHUNK)
    cols3 = jnp.concatenate(
        [col, jnp.zeros((pad_n,), jnp.int32)]).reshape(NS, NCHUNK, CHUNK)
    zeros_h = jnp.zeros((RPT, HHID), jnp.float32)
    zeros16 = jnp.zeros((RPT, 16), jnp.float32)
    ones16 = jnp.ones((CHUNK, 16), jnp.float32)

    beta = jnp.asarray(BETAS, jnp.float32)
    Wb = ALPHA * beta[:, None, None] * Wl
    sgb = jnp.broadcast_to((ALPHA * (1.0 - beta))[:, None, None],
                           (NUM_LAYERS, 1, HID))
    bb = (beta[:, None] * bl)[:, None, :]

    deg_p = _sc_degree(rows3, ones16, zeros16)
    dis = _tc_dis(deg_p)
    g = _tc_prep(x, W0, b0)

    u = _tc_u0(g, dis, Wl[0], BETAS[0])
    P = _sc_spmm(u, rows3, cols3, zeros_h)
    Cstack = _tc_cterms(g, Wb, sgb, bb)
    Cs = [Cstack[i] for i in range(NUM_LAYERS)]
    for i in range(NUM_LAYERS):
        if i > 0:
            P = _sc_spmm(u, rows3, cols3, zeros_h)
        if i + 1 < NUM_LAYERS:
            u = _tc_layer(P, dis, Cs[i], Wl[i + 1], BETAS[i + 1])
        else:
            return _tc_final(P, dis, Cs[i], W1, b1)
